# baseline (device time: 7433 ns/iter reference)
import jax
import jax.numpy as jnp
from jax import lax
from jax.experimental import pallas as pl
from jax.experimental.pallas import tpu as pltpu

N_BLK = 6


def kernel(x):
    m, n = x.shape
    rows, lanes = m // 128, 128
    bm = m // N_BLK
    brows = bm // 128
    half = rows // 2

    def body(x_ref, out_ref, comm_ref, send_sems, recv_sems):
        i = pl.program_id(0)
        my_x = lax.axis_index("x")
        my_y = lax.axis_index("y")
        nbr = (my_x, 1 - my_y)

        barrier_sem = pltpu.get_barrier_semaphore()

        @pl.when(i == 0)
        def _():
            pl.semaphore_signal(
                barrier_sem, inc=1, device_id=nbr,
                device_id_type=pl.DeviceIdType.MESH,
            )

        partial = jnp.max(x_ref[:, :], axis=1)
        comm_ref[0, pl.ds(i * brows, brows), :] = jnp.reshape(
            partial, (brows, lanes)
        )

        def make_rdma(sl, k):
            return pltpu.make_async_remote_copy(
                src_ref=comm_ref.at[0, sl],
                dst_ref=comm_ref.at[1, sl],
                send_sem=send_sems.at[k],
                recv_sem=recv_sems.at[k],
                device_id=nbr,
                device_id_type=pl.DeviceIdType.MESH,
            )

        @pl.when(i == N_BLK // 2 - 1)
        def _():
            pl.semaphore_wait(barrier_sem, 1)
            make_rdma(pl.ds(0, half), 0).start()

        @pl.when(i == N_BLK - 1)
        def _():
            make_rdma(pl.ds(half, rows - half), 1).start()
            make_rdma(pl.ds(0, half), 0).wait()
            make_rdma(pl.ds(half, rows - half), 1).wait()
            out_ref[:, :] = jnp.maximum(comm_ref[0, :, :], comm_ref[1, :, :])

    packed = pl.pallas_call(
        body,
        grid=(N_BLK,),
        out_shape=jax.ShapeDtypeStruct((rows, lanes), jnp.float32),
        in_specs=[
            pl.BlockSpec((bm, n), lambda i: (i, 0), memory_space=pltpu.VMEM)
        ],
        out_specs=pl.BlockSpec(
            (rows, lanes), lambda i: (0, 0), memory_space=pltpu.VMEM
        ),
        scratch_shapes=[
            pltpu.VMEM((2, rows, lanes), jnp.float32),
            pltpu.SemaphoreType.DMA((2,)),
            pltpu.SemaphoreType.DMA((2,)),
        ],
        compiler_params=pltpu.CompilerParams(collective_id=0),
    )(x)
    return jnp.reshape(packed, (m, 1))


# device time: 6954 ns/iter; 1.0689x vs baseline; 1.0689x over previous
import jax
import jax.numpy as jnp
from jax import lax
from jax.experimental import pallas as pl
from jax.experimental.pallas import tpu as pltpu


def kernel(x):
    m, n = x.shape
    rows, lanes = m // 128, 128
    half = rows // 2
    mh = m // 2

    def body(x_ref, out_ref, comm_ref, send_sems, recv_sems):
        my_x = lax.axis_index("x")
        my_y = lax.axis_index("y")
        nbr = (my_x, 1 - my_y)

        barrier_sem = pltpu.get_barrier_semaphore()
        pl.semaphore_signal(
            barrier_sem, inc=1, device_id=nbr,
            device_id_type=pl.DeviceIdType.MESH,
        )

        def make_rdma(sl, k):
            return pltpu.make_async_remote_copy(
                src_ref=comm_ref.at[0, sl],
                dst_ref=comm_ref.at[1, sl],
                send_sem=send_sems.at[k],
                recv_sem=recv_sems.at[k],
                device_id=nbr,
                device_id_type=pl.DeviceIdType.MESH,
            )

        lo = pl.ds(0, half)
        hi = pl.ds(half, rows - half)

        pa = jnp.max(x_ref[pl.ds(0, mh), :], axis=1)
        comm_ref[0, lo, :] = jnp.reshape(pa, (half, lanes))
        pl.semaphore_wait(barrier_sem, 1)
        rdma0 = make_rdma(lo, 0)
        rdma0.start()

        pb = jnp.max(x_ref[pl.ds(mh, mh), :], axis=1)
        comm_ref[0, hi, :] = jnp.reshape(pb, (rows - half, lanes))
        rdma1 = make_rdma(hi, 1)
        rdma1.start()

        rdma0.wait()
        out_ref[lo, :] = jnp.maximum(comm_ref[0, lo, :], comm_ref[1, lo, :])
        rdma1.wait()
        out_ref[hi, :] = jnp.maximum(comm_ref[0, hi, :], comm_ref[1, hi, :])

    packed = pl.pallas_call(
        body,
        out_shape=jax.ShapeDtypeStruct((rows, lanes), jnp.float32),
        in_specs=[pl.BlockSpec(memory_space=pltpu.VMEM)],
        out_specs=pl.BlockSpec(memory_space=pltpu.VMEM),
        scratch_shapes=[
            pltpu.VMEM((2, rows, lanes), jnp.float32),
            pltpu.SemaphoreType.DMA((2,)),
            pltpu.SemaphoreType.DMA((2,)),
        ],
        compiler_params=pltpu.CompilerParams(collective_id=0),
    )(x)
    return jnp.reshape(packed, (m, 1))
